# baseline (speedup 1.0000x reference)
import functools

import jax
import jax.numpy as jnp
from jax import lax
from jax.experimental import pallas as pl
from jax.experimental.pallas import tpu as pltpu
from jax.experimental.pallas import tpu_sc as plsc

NC = 2
NS = 16
C = 125


def _combine_matmul(summed, W):
    n, d = summed.shape
    d_out = W.shape[0]
    blk = 2000
    dn = (((1,), (1,)), ((), ()))

    def body(p_ref, w_ref, o_ref):
        o_ref[...] = lax.dot_general(p_ref[...], w_ref[...], dn,
                                     preferred_element_type=jnp.float32)

    return pl.pallas_call(
        body,
        grid=(n // blk,),
        in_specs=[
            pl.BlockSpec((blk, d), lambda i: (i, 0)),
            pl.BlockSpec(W.shape, lambda i: (0, 0)),
        ],
        out_specs=pl.BlockSpec((blk, d_out), lambda i: (i, 0)),
        out_shape=jax.ShapeDtypeStruct((n, d_out), jnp.float32),
    )(summed, W)


def _make_sc_scatter(n, hd, ch_per_tile):
    rows_per_tile = n // NS
    mesh = plsc.VectorSubcoreMesh(
        core_axis_name="c", subcore_axis_name="s",
        num_cores=NC, num_subcores=NS)

    @functools.partial(
        pl.kernel,
        out_type=jax.ShapeDtypeStruct((n, NC * hd), jnp.float32),
        mesh=mesh,
        compiler_params=pltpu.CompilerParams(use_tc_tiling_on_sc=False),
        scratch_types=(
            [pltpu.VMEM((ch_per_tile, C), jnp.int32)] * 2
            + [pltpu.VMEM((C, hd), jnp.float32)] * 6
            + [pltpu.VMEM_SHARED((n, hd), jnp.float32)]
            + [pltpu.SemaphoreType.DMA] * 12
        ),
    )
    def sc_scatter(x2_hbm, idx_hbm, zeros_hbm, out_hbm,
                   src_v, tgt_v, b0, b1, b2, b3, b4, b5, acc,
                   g0, g1, g2, g3, g4, g5,
                   s0, s1, s2, s3, s4, s5):
        bufs = (b0, b1, b2, b3, b4, b5)
        gsems = (g0, g1, g2, g3, g4, g5)
        ssems = (s0, s1, s2, s3, s4, s5)
        c = lax.axis_index("c")
        s = lax.axis_index("s")

        def gfire(j, k):
            pltpu.async_copy(x2_hbm.at[src_v.at[j]], bufs[k], gsems[k])

        def gwait(j, k):
            pltpu.make_async_copy(
                x2_hbm.at[src_v.at[j]], bufs[k], gsems[k]).wait()

        def sfire(j, k):
            pltpu.async_copy(bufs[k], acc.at[tgt_v.at[j]], ssems[k], add=True)

        def swait(j, k):
            pltpu.make_async_copy(
                bufs[k], acc.at[tgt_v.at[j]], ssems[k]).wait()

        n_main = (ch_per_tile // 6) * 6 if ch_per_tile >= 6 else 0

        def ring_body(g, carry):
            for k in range(6):
                j = 6 * g + k
                gwait(j, k)
                sfire(j, k)
                k2 = (k + 3) % 6
                if k < 3:
                    @pl.when(g > 0)
                    def _():
                        swait(j - 3, k2)
                else:
                    swait(j - 3, k2)

                @pl.when(j + 3 < ch_per_tile)
                def _():
                    gfire(j + 3, k2)
            return carry

        pltpu.sync_copy(idx_hbm.at[c, s], src_v)
        pltpu.sync_copy(idx_hbm.at[NC, s], tgt_v)
        if n_main:
            for jp in range(3):
                gfire(jp, jp)
        off = 0
        while off < rows_per_tile:
            m = min(C, rows_per_tile - off)
            pltpu.sync_copy(zeros_hbm.at[pl.ds(0, m)],
                            acc.at[pl.ds(s * rows_per_tile + off, m)])
            off += m
        plsc.subcore_barrier()

        if n_main:
            lax.fori_loop(0, n_main // 6, ring_body, 0)
            drained = n_main - 4
            for j in range(n_main, ch_per_tile):
                gwait(j, j % 6)
                sfire(j, j % 6)
                if j - 3 > drained:
                    swait(j - 3, (j - 3) % 6)
                    drained = j - 3
                if j + 3 < ch_per_tile:
                    gfire(j + 3, (j + 3) % 6)
            for j in range(drained + 1, ch_per_tile):
                swait(j, j % 6)
        else:
            for j in range(ch_per_tile):
                gfire(j, 0)
                gwait(j, 0)
                pltpu.sync_copy(bufs[0], acc.at[tgt_v.at[j]], add=True)
        plsc.subcore_barrier()
        pltpu.sync_copy(
            acc.at[pl.ds(s * rows_per_tile, rows_per_tile)],
            out_hbm.at[pl.ds(s * rows_per_tile, rows_per_tile),
                       pl.ds(c * hd, hd)])

    return sc_scatter


def kernel(x, source, target, num_nodes, W):
    del num_nodes
    n, d = x.shape
    e = source.shape[0]
    hd = d // NC
    ch_per_tile = e // (C * NS)

    x2 = x.reshape(NC * n, hd)
    src32 = source.astype(jnp.int32)
    tgt32 = target.astype(jnp.int32)
    idx_all = jnp.concatenate(
        [NC * src32[None, :] + jnp.arange(NC, dtype=jnp.int32)[:, None],
         tgt32[None, :]], axis=0).reshape(NC + 1, NS, ch_per_tile, C)
    zeros = jnp.zeros((C, hd), jnp.float32)
    sc_scatter = _make_sc_scatter(n, hd, ch_per_tile)
    summed = sc_scatter(x2, idx_all, zeros)
    return _combine_matmul(summed, W)


# speedup vs baseline: 1.0702x; 1.0702x over previous
import functools

import jax
import jax.numpy as jnp
from jax import lax
from jax.experimental import pallas as pl
from jax.experimental.pallas import tpu as pltpu
from jax.experimental.pallas import tpu_sc as plsc

NC = 2
NS = 16
C = 125


def _combine_matmul(summed, W):
    n, d = summed.shape
    d_out = W.shape[0]
    blk = 2000
    dn = (((1,), (1,)), ((), ()))

    def body(p_ref, w_ref, o_ref):
        o_ref[...] = lax.dot_general(p_ref[...], w_ref[...], dn,
                                     preferred_element_type=jnp.float32)

    return pl.pallas_call(
        body,
        grid=(n // blk,),
        in_specs=[
            pl.BlockSpec((blk, d), lambda i: (i, 0)),
            pl.BlockSpec(W.shape, lambda i: (0, 0)),
        ],
        out_specs=pl.BlockSpec((blk, d_out), lambda i: (i, 0)),
        out_shape=jax.ShapeDtypeStruct((n, d_out), jnp.float32),
    )(summed, W)


def _make_sc_scatter(n, hd, ch_per_tile):
    rows_per_tile = n // NS
    mesh = plsc.VectorSubcoreMesh(
        core_axis_name="c", subcore_axis_name="s",
        num_cores=NC, num_subcores=NS)

    @functools.partial(
        pl.kernel,
        out_type=jax.ShapeDtypeStruct((n, NC * hd), jnp.float32),
        mesh=mesh,
        compiler_params=pltpu.CompilerParams(use_tc_tiling_on_sc=False),
        scratch_types=(
            [pltpu.VMEM((ch_per_tile, C), jnp.int32)] * 2
            + [pltpu.VMEM((C, hd), jnp.float32)] * 6
            + [pltpu.VMEM_SHARED((n, hd), jnp.float32)]
            + [pltpu.SemaphoreType.DMA] * 12
        ),
    )
    def sc_scatter(x2_hbm, src_hbm, tgt_hbm, out_hbm,
                   src_v, tgt_v, b0, b1, b2, b3, b4, b5, acc,
                   g0, g1, g2, g3, g4, g5,
                   s0, s1, s2, s3, s4, s5):
        bufs = (b0, b1, b2, b3, b4, b5)
        gsems = (g0, g1, g2, g3, g4, g5)
        ssems = (s0, s1, s2, s3, s4, s5)
        c = lax.axis_index("c")
        s = lax.axis_index("s")

        def gfire(j, k):
            pltpu.async_copy(x2_hbm.at[src_v.at[j]], bufs[k], gsems[k])

        def gwait(j, k):
            pltpu.make_async_copy(
                x2_hbm.at[src_v.at[j]], bufs[k], gsems[k]).wait()

        def sfire(j, k):
            pltpu.async_copy(bufs[k], acc.at[tgt_v.at[j]], ssems[k], add=True)

        def swait(j, k):
            pltpu.make_async_copy(
                bufs[k], acc.at[tgt_v.at[j]], ssems[k]).wait()

        n_main = (ch_per_tile // 6) * 6 if ch_per_tile >= 6 else 0

        def ring_body(g, carry):
            for k in range(6):
                j = 6 * g + k
                gwait(j, k)
                sfire(j, k)
                k2 = (k + 3) % 6
                if k < 3:
                    @pl.when(g > 0)
                    def _():
                        swait(j - 3, k2)
                else:
                    swait(j - 3, k2)

                @pl.when(j + 3 < ch_per_tile)
                def _():
                    gfire(j + 3, k2)
            return carry

        pltpu.sync_copy(src_hbm.at[c, s], src_v)
        pltpu.sync_copy(tgt_hbm.at[s], tgt_v)
        if n_main:
            for jp in range(3):
                gfire(jp, jp)
        zero16 = jnp.zeros((16,), jnp.float32)

        def zero_body(i, carry):
            for k in range(hd // 16):
                b5[i, pl.ds(k * 16, 16)] = zero16
            return carry

        lax.fori_loop(0, C, zero_body, 0)
        off = 0
        while off < rows_per_tile:
            m = min(C, rows_per_tile - off)
            pltpu.sync_copy(b5.at[pl.ds(0, m)],
                            acc.at[pl.ds(s * rows_per_tile + off, m)])
            off += m
        plsc.subcore_barrier()

        if n_main:
            lax.fori_loop(0, n_main // 6, ring_body, 0)
            drained = n_main - 4
            for j in range(n_main, ch_per_tile):
                gwait(j, j % 6)
                sfire(j, j % 6)
                if j - 3 > drained:
                    swait(j - 3, (j - 3) % 6)
                    drained = j - 3
                if j + 3 < ch_per_tile:
                    gfire(j + 3, (j + 3) % 6)
            for j in range(drained + 1, ch_per_tile):
                swait(j, j % 6)
        else:
            for j in range(ch_per_tile):
                gfire(j, 0)
                gwait(j, 0)
                pltpu.sync_copy(bufs[0], acc.at[tgt_v.at[j]], add=True)
        plsc.subcore_barrier()
        pltpu.sync_copy(
            acc.at[pl.ds(s * rows_per_tile, rows_per_tile)],
            out_hbm.at[pl.ds(s * rows_per_tile, rows_per_tile),
                       pl.ds(c * hd, hd)])

    return sc_scatter


def kernel(x, source, target, num_nodes, W):
    del num_nodes
    n, d = x.shape
    e = source.shape[0]
    hd = d // NC
    ch_per_tile = e // (C * NS)

    x2 = x.reshape(NC * n, hd)
    src32 = source.astype(jnp.int32)
    src_both = (NC * src32[None, :]
                + jnp.arange(NC, dtype=jnp.int32)[:, None]
                ).reshape(NC, NS, ch_per_tile, C)
    tgt3 = target.reshape(NS, ch_per_tile, C).astype(jnp.int32)
    sc_scatter = _make_sc_scatter(n, hd, ch_per_tile)
    summed = sc_scatter(x2, src_both, tgt3)
    return _combine_matmul(summed, W)
